# Initial kernel scaffold; baseline (speedup 1.0000x reference)
import jax
import jax.numpy as jnp
from jax.experimental import pallas as pl


def kernel(mem, idx, val):
    raise NotImplementedError("write your pallas kernel here")


# speedup vs baseline: 3.7538x; 3.7538x over previous
import functools

import jax
import jax.numpy as jnp
from jax import lax
from jax.experimental import pallas as pl
from jax.experimental.pallas import tpu as pltpu
from jax.experimental.pallas import tpu_sc as plsc

T = 8
G3 = 128 ** 3
N = T * G3
U = T * (G3 // 4)
DECAY = 0.95

NC, NS = 2, 16
NW = NC * NS
CHUNK = 65536
NCHUNK = N // CHUNK
CPT = NCHUNK // NW
BATCH = 2048
KPAD = BATCH + 64
SENTINEL = 0x7FFFFFFF
NBOUND = NCHUNK + 1
BPAD = 288

_MESH = plsc.VectorSubcoreMesh(core_axis_name="c", subcore_axis_name="s")


@functools.partial(
    pl.kernel,
    mesh=_MESH,
    compiler_params=pltpu.CompilerParams(needs_layout_passes=False),
    out_type=jax.ShapeDtypeStruct((N,), jnp.float32),
    scratch_types=[
        pltpu.VMEM((CHUNK,), jnp.float32),
        pltpu.VMEM((KPAD,), jnp.int32),
        pltpu.VMEM((KPAD,), jnp.float32),
        pltpu.VMEM((BPAD,), jnp.int32),
    ],
)
def _grid_update(mem_hbm, sk_hbm, sv_hbm, bnd_hbm, out_hbm,
                 buf, kbuf, vbuf, bvec):
    wid = lax.axis_index("s") * NC + lax.axis_index("c")
    pltpu.sync_copy(bnd_hbm, bvec)
    lanes = lax.iota(jnp.int32, 16)

    def _bnd(pos):
        return plsc.load_gather(bvec, [jnp.zeros((16,), jnp.int32) + pos])[0]

    def chunk_body(j, _):
        k = wid * CPT + j
        cb = k * CHUNK
        pltpu.sync_copy(mem_hbm.at[pl.ds(cb, CHUNK)], buf)
        s = _bnd(k)
        e = _bnd(k + 1)
        astart = (s // 16) * 16
        nb = (e - astart + (BATCH - 1)) // BATCH

        def batch_body(ib, _):
            base = astart + ib * BATCH
            pltpu.sync_copy(sk_hbm.at[pl.ds(base, KPAD)], kbuf)
            pltpu.sync_copy(sv_hbm.at[pl.ds(base, KPAD)], vbuf)

            def vec_body(i, _):
                off = i * 16
                ka = kbuf[pl.ds(off, 16)]
                kb = plsc.load_gather(kbuf, [off + 1 + lanes])
                v = vbuf[pl.ds(off, 16)]
                pos = (base + off) + lanes
                m = (ka != kb) & (pos >= s) & (pos < e)
                lidx = jnp.minimum(jnp.maximum(ka - cb, 0), CHUNK - 1)
                g = plsc.load_gather(buf, [lidx])
                w = jnp.maximum(g * jnp.float32(DECAY), v)
                plsc.store_scatter(buf, [lidx], w, mask=m)
                return 0

            lax.fori_loop(0, BATCH // 16, vec_body, 0)
            return 0

        lax.fori_loop(0, nb, batch_body, 0)
        pltpu.sync_copy(buf, out_hbm.at[pl.ds(cb, CHUNK)])
        return 0

    lax.fori_loop(0, CPT, chunk_body, 0)


def kernel(mem, idx, val):
    keys = (idx.astype(jnp.int32)
            + (jnp.arange(T, dtype=jnp.int32) * G3)[:, None]).reshape(-1)
    sk, sv = lax.sort((keys, val.reshape(-1)), is_stable=False, num_keys=1)
    bnd = jnp.searchsorted(
        sk, jnp.arange(NBOUND, dtype=jnp.int32) * CHUNK, side="left"
    ).astype(jnp.int32)
    bnd = jnp.concatenate([bnd, jnp.zeros((BPAD - NBOUND,), jnp.int32)])
    skp = jnp.concatenate([sk, jnp.full((KPAD,), SENTINEL, jnp.int32)])
    svp = jnp.concatenate([sv, jnp.zeros((KPAD,), jnp.float32)])
    out = _grid_update(mem.reshape(-1), skp, svp, bnd)
    return out.reshape(T, G3)
